# baseline (speedup 1.0000x reference)
import functools

import jax
import jax.numpy as jnp
from jax import lax
from jax.experimental import pallas as pl
from jax.experimental.pallas import tpu as pltpu
from jax.experimental.pallas import tpu_sc as plsc

B = 16384
V = 1000000
D = 64
DP = 128
NEG = 5

NC = 2
NS = 16
L = 16
NW = NC * NS
NB = B // NW
CH = 16
NCHUNK = NB // CH
KD = D // L


def _sc_scores():
    mesh = plsc.VectorSubcoreMesh(
        core_axis_name="c", subcore_axis_name="s", num_cores=NC, num_subcores=NS
    )

    @functools.partial(
        pl.kernel,
        mesh=mesh,
        compiler_params=pltpu.CompilerParams(needs_layout_passes=False),
        out_type=[
            jax.ShapeDtypeStruct((B,), jnp.float32),
            jax.ShapeDtypeStruct((B * NEG,), jnp.float32),
        ],
        scratch_types=[
            pltpu.VMEM((NB,), jnp.int32),
            pltpu.VMEM((NB,), jnp.int32),
            pltpu.VMEM((NB,), jnp.float32),
            pltpu.VMEM((NB,), jnp.int32),
            pltpu.VMEM((NB * NEG,), jnp.int32),
            pltpu.VMEM((CH, DP), jnp.float32),
            pltpu.VMEM((CH, DP), jnp.float32),
            pltpu.VMEM((CH * NEG, DP), jnp.float32),
            pltpu.VMEM((NB,), jnp.float32),
            pltpu.VMEM((NB * NEG,), jnp.float32),
            pltpu.SemaphoreType.DMA,
            pltpu.SemaphoreType.DMA,
            pltpu.SemaphoreType.DMA,
        ],
    )
    def body(pos_w_hbm, pos_v_hbm, neg_hbm, wp_hbm, vp_hbm,
             pos_out_hbm, neg_out_hbm,
             idx_w, half_w, par_w, idx_v, idx_n, rows_w, rows_v, rows_n,
             pos_buf, neg_buf, sem_w, sem_v, sem_n):
        wid = lax.axis_index("s") * NC + lax.axis_index("c")
        base = wid * NB
        nbase = wid * NB * NEG

        pltpu.sync_copy(pos_w_hbm.at[pl.ds(base, NB)], idx_w)
        pltpu.sync_copy(pos_v_hbm.at[pl.ds(base, NB)], idx_v)
        pltpu.sync_copy(neg_hbm.at[pl.ds(nbase, NB * NEG)], idx_n)

        lane = lax.iota(jnp.int32, 16)

        def split(t, _):
            x = idx_w[pl.ds(t * L, L)]
            half_w[pl.ds(t * L, L)] = x >> 1
            par_w[pl.ds(t * L, L)] = (x & 1).astype(jnp.float32)
            return 0

        lax.fori_loop(0, NB // L, split, 0)

        def chunk(c, _):
            gw = pltpu.async_copy(
                wp_hbm.at[half_w.at[pl.ds(c * CH, CH)]], rows_w, sem_w)
            gv = pltpu.async_copy(
                vp_hbm.at[idx_v.at[pl.ds(c * CH, CH)]], rows_v, sem_v)
            gn = pltpu.async_copy(
                vp_hbm.at[idx_n.at[pl.ds(c * CH * NEG, CH * NEG)]],
                rows_n, sem_n)
            gw.wait()
            gv.wait()
            gn.wait()

            accp = jnp.zeros((16,), jnp.float32)
            accn = [jnp.zeros((16,), jnp.float32) for _ in range(NEG)]
            pw = par_w[pl.ds(c * CH, CH)]
            for b in range(CH):
                vv = [rows_v[b, pl.ds(k * L, L)] for k in range(KD)]
                sp = jnp.take(pw, jnp.full((16,), b, jnp.int32))

                def wrow(k):
                    a0 = rows_w[b, pl.ds(k * L, L)]
                    a1 = rows_w[b, pl.ds(D + k * L, L)]
                    return a0 + (a1 - a0) * sp

                p = wrow(0) * vv[0]
                for k in range(1, KD):
                    p = p + wrow(k) * vv[k]
                accp = jnp.where(lane == b, jnp.sum(p), accp)
                for n in range(NEG):
                    r = b * NEG + n
                    q = rows_n[r, pl.ds(0, L)] * vv[0]
                    for k in range(1, KD):
                        q = q + rows_n[r, pl.ds(k * L, L)] * vv[k]
                    accn[n] = jnp.where(lane == b, jnp.sum(q), accn[n])

            pos_buf[pl.ds(c * CH, CH)] = accp
            for n in range(NEG):
                neg_buf[pl.ds(n * NB + c * CH, CH)] = accn[n]
            return 0

        lax.fori_loop(0, NCHUNK, chunk, 0)

        pltpu.sync_copy(pos_buf, pos_out_hbm.at[pl.ds(base, NB)])
        pltpu.sync_copy(neg_buf, neg_out_hbm.at[pl.ds(nbase, NB * NEG)])

    return body


_SC_SCORES = _sc_scores()


def _tc_transpad_body(x_ref, o_ref):
    t = x_ref[...].T
    o_ref[:, :D] = t
    o_ref[:, D:] = jnp.zeros_like(t)


def _tc_transpad(tT):
    CB = 4096
    return pl.pallas_call(
        _tc_transpad_body,
        grid=(pl.cdiv(V, CB),),
        in_specs=[pl.BlockSpec((D, CB), lambda c: (0, c))],
        out_specs=pl.BlockSpec((CB, DP), lambda c: (c, 0)),
        out_shape=jax.ShapeDtypeStruct((V, DP), jnp.float32),
    )(tT)


def _tc_loss_body(p_ref, n_ref, o_ref):
    p = p_ref[...]
    n = -n_ref[...]
    lsp = jnp.minimum(p, 0.0) - jnp.log1p(jnp.exp(-jnp.abs(p)))
    lsn = jnp.minimum(n, 0.0) - jnp.log1p(jnp.exp(-jnp.abs(n)))
    o_ref[0, 0] = -(jnp.sum(lsp) + jnp.sum(lsn))


def kernel(pos_w, pos_v, neg_v, w_emb, v_emb):
    pos_w = pos_w.astype(jnp.int32)
    pos_v = pos_v.astype(jnp.int32)
    neg_flat = neg_v.reshape(-1).astype(jnp.int32)
    wp = w_emb.reshape(V // 2, 2 * D)
    vp = _tc_transpad(v_emb.T)

    pos_raw, neg_raw = _SC_SCORES(pos_w, pos_v, neg_flat, wp, vp)

    loss = pl.pallas_call(
        _tc_loss_body,
        out_shape=jax.ShapeDtypeStruct((1, 1), jnp.float32),
        out_specs=pl.BlockSpec(memory_space=pltpu.SMEM),
    )(pos_raw.reshape(B // 128, 128), neg_raw.reshape(B * NEG // 128, 128))
    return loss[0, 0]


# speedup vs baseline: 1.0833x; 1.0833x over previous
import functools

import jax
import jax.numpy as jnp
from jax import lax
from jax.experimental import pallas as pl
from jax.experimental.pallas import tpu as pltpu
from jax.experimental.pallas import tpu_sc as plsc

B = 16384
V = 1000000
D = 64
DP = 128
NEG = 5

NC = 2
NS = 16
L = 16
NW = NC * NS
NB = B // NW
CH = 16
NCHUNK = NB // CH
KD = D // L


def _sc_scores():
    mesh = plsc.VectorSubcoreMesh(
        core_axis_name="c", subcore_axis_name="s", num_cores=NC, num_subcores=NS
    )

    @functools.partial(
        pl.kernel,
        mesh=mesh,
        compiler_params=pltpu.CompilerParams(needs_layout_passes=False),
        out_type=[
            jax.ShapeDtypeStruct((B,), jnp.float32),
            jax.ShapeDtypeStruct((B * NEG,), jnp.float32),
        ],
        scratch_types=[
            pltpu.VMEM((NB,), jnp.int32),
            pltpu.VMEM((NB,), jnp.int32),
            pltpu.VMEM((NB * NEG,), jnp.int32),
            pltpu.VMEM((CH, DP), jnp.float32),
            pltpu.VMEM((CH, DP), jnp.float32),
            pltpu.VMEM((CH * NEG, DP), jnp.float32),
            pltpu.VMEM((NB,), jnp.float32),
            pltpu.VMEM((NB * NEG,), jnp.float32),
            pltpu.SemaphoreType.DMA,
            pltpu.SemaphoreType.DMA,
            pltpu.SemaphoreType.DMA,
        ],
    )
    def body(pos_w_hbm, pos_v_hbm, neg_hbm, wp_hbm, vp_hbm,
             pos_out_hbm, neg_out_hbm,
             idx_w, idx_v, idx_n, rows_w, rows_v, rows_n,
             pos_buf, neg_buf, sem_w, sem_v, sem_n):
        wid = lax.axis_index("s") * NC + lax.axis_index("c")
        base = wid * NB
        nbase = wid * NB * NEG

        pltpu.sync_copy(pos_w_hbm.at[pl.ds(base, NB)], idx_w)
        pltpu.sync_copy(pos_v_hbm.at[pl.ds(base, NB)], idx_v)
        pltpu.sync_copy(neg_hbm.at[pl.ds(nbase, NB * NEG)], idx_n)

        lane = lax.iota(jnp.int32, 16)

        def chunk(c, _):
            gw = pltpu.async_copy(
                wp_hbm.at[idx_w.at[pl.ds(c * CH, CH)]], rows_w, sem_w)
            gv = pltpu.async_copy(
                vp_hbm.at[idx_v.at[pl.ds(c * CH, CH)]], rows_v, sem_v)
            gn = pltpu.async_copy(
                vp_hbm.at[idx_n.at[pl.ds(c * CH * NEG, CH * NEG)]],
                rows_n, sem_n)
            gw.wait()
            gv.wait()
            gn.wait()

            accp = jnp.zeros((16,), jnp.float32)
            accn = [jnp.zeros((16,), jnp.float32) for _ in range(NEG)]
            for b in range(CH):
                vv = [rows_v[b, pl.ds(k * L, L)] for k in range(KD)]
                p = rows_w[b, pl.ds(0, L)] * vv[0]
                for k in range(1, KD):
                    p = p + rows_w[b, pl.ds(k * L, L)] * vv[k]
                accp = jnp.where(lane == b, jnp.sum(p), accp)
                for n in range(NEG):
                    r = b * NEG + n
                    q = rows_n[r, pl.ds(0, L)] * vv[0]
                    for k in range(1, KD):
                        q = q + rows_n[r, pl.ds(k * L, L)] * vv[k]
                    accn[n] = jnp.where(lane == b, jnp.sum(q), accn[n])

            pos_buf[pl.ds(c * CH, CH)] = accp
            for n in range(NEG):
                neg_buf[pl.ds(n * NB + c * CH, CH)] = accn[n]
            return 0

        lax.fori_loop(0, NCHUNK, chunk, 0)

        pltpu.sync_copy(pos_buf, pos_out_hbm.at[pl.ds(base, NB)])
        pltpu.sync_copy(neg_buf, neg_out_hbm.at[pl.ds(nbase, NB * NEG)])

    return body


_SC_SCORES = _sc_scores()


def _tc_transpad_body(x_ref, o_ref):
    t = x_ref[...].T
    o_ref[:, :D] = t
    o_ref[:, D:] = jnp.zeros_like(t)


def _tc_transpad(tT):
    CB = 4096
    return pl.pallas_call(
        _tc_transpad_body,
        grid=(pl.cdiv(V, CB),),
        in_specs=[pl.BlockSpec((D, CB), lambda c: (0, c))],
        out_specs=pl.BlockSpec((CB, DP), lambda c: (c, 0)),
        out_shape=jax.ShapeDtypeStruct((V, DP), jnp.float32),
    )(tT)


def _tc_loss_body(p_ref, n_ref, o_ref):
    p = p_ref[...]
    n = -n_ref[...]
    lsp = jnp.minimum(p, 0.0) - jnp.log1p(jnp.exp(-jnp.abs(p)))
    lsn = jnp.minimum(n, 0.0) - jnp.log1p(jnp.exp(-jnp.abs(n)))
    o_ref[0, 0] = -(jnp.sum(lsp) + jnp.sum(lsn))


def kernel(pos_w, pos_v, neg_v, w_emb, v_emb):
    pos_w = pos_w.astype(jnp.int32)
    pos_v = pos_v.astype(jnp.int32)
    neg_flat = neg_v.reshape(-1).astype(jnp.int32)
    wp = jnp.pad(w_emb, ((0, 0), (0, DP - D)))
    vp = _tc_transpad(v_emb.T)

    pos_raw, neg_raw = _SC_SCORES(pos_w, pos_v, neg_flat, wp, vp)

    loss = pl.pallas_call(
        _tc_loss_body,
        out_shape=jax.ShapeDtypeStruct((1, 1), jnp.float32),
        out_specs=pl.BlockSpec(memory_space=pltpu.SMEM),
    )(pos_raw.reshape(B // 128, 128), neg_raw.reshape(B * NEG // 128, 128))
    return loss[0, 0]
